# baseline (speedup 1.0000x reference)
import functools

import jax
import jax.numpy as jnp
from jax import lax
from jax.experimental import pallas as pl
from jax.experimental.pallas import tpu as pltpu
from jax.experimental.pallas import tpu_sc as plsc

N = 10000
E = 320000
D = 128
G = 64
RATIO = 0.5

NPAD = 10240
NC = 2
NT = 16
ET = E // (NC * NT)
W = 80
NWIN = ET // W
ROWS_PER_TILE = NPAD // NT

BR = 1280
NBLK = NPAD // BR

_mesh = plsc.VectorSubcoreMesh(core_axis_name="c", subcore_axis_name="s",
                               num_cores=NC)


def _zero16():
    return jnp.zeros((16,), jnp.float32)


def _sc_mask_deg_body(src_h, dst_h, ew_h, kf_h, ewm_h, degp_h,
                      kf_v, sidx_all, didx_all, ew_all, ewm_all, zb, acc, sem):
    cid = lax.axis_index("c")
    sid = lax.axis_index("s")
    z16 = _zero16()
    for i in range(40):
        zb[pl.ds(i * 16, 16)] = z16
    pltpu.sync_copy(zb, acc.at[pl.ds(sid * 640, 640)])
    pltpu.sync_copy(kf_h, kf_v)
    wbase = (cid * NT + sid) * NWIN
    pltpu.sync_copy(src_h.at[pl.ds(wbase, NWIN)], sidx_all)
    pltpu.sync_copy(dst_h.at[pl.ds(wbase, NWIN)], didx_all)
    pltpu.sync_copy(ew_h.at[pl.ds(wbase, NWIN)], ew_all)
    plsc.subcore_barrier()

    def win(w, carry):
        for k in range(W // 16):
            sv = sidx_all[w, 0, pl.ds(k * 16, 16)]
            dv = didx_all[w, 0, pl.ds(k * 16, 16)]
            ks = plsc.load_gather(kf_v, [sv])
            kd = plsc.load_gather(kf_v, [dv])
            ewm_all[w, 0, pl.ds(k * 16, 16)] = \
                ew_all[w, 0, pl.ds(k * 16, 16)] * ks * kd
        return carry

    lax.fori_loop(0, NWIN, win, 0)
    pltpu.sync_copy(ewm_all, ewm_h.at[pl.ds(wbase, NWIN)])

    def sc(w, carry):
        pltpu.sync_copy(ewm_all.at[w, 0], acc.at[didx_all.at[w, 0]],
                        add=True)
        return carry

    lax.fori_loop(0, NWIN, sc, 0)
    plsc.subcore_barrier()
    pltpu.sync_copy(acc.at[pl.ds(sid * 640, 640)],
                    degp_h.at[cid, pl.ds(sid * 640, 640)])


_sc_mask_deg = functools.partial(
    pl.kernel,
    out_type=[jax.ShapeDtypeStruct((E // W, 1, W), jnp.float32),
              jax.ShapeDtypeStruct((NC, NPAD), jnp.float32)],
    mesh=_mesh,
    compiler_params=pltpu.CompilerParams(needs_layout_passes=False),
    scratch_types=[
        pltpu.VMEM((NPAD,), jnp.float32),
        pltpu.VMEM((NWIN, 1, W), jnp.int32),
        pltpu.VMEM((NWIN, 1, W), jnp.int32),
        pltpu.VMEM((NWIN, 1, W), jnp.float32),
        pltpu.VMEM((NWIN, 1, W), jnp.float32),
        pltpu.VMEM((640,), jnp.float32),
        pltpu.VMEM_SHARED((NPAD,), jnp.float32),
        pltpu.SemaphoreType.DMA,
    ],
)(_sc_mask_deg_body)


SW = 40
NW2 = ET // SW


def _sc_prop_body(hwp_h, src_h, dst_h, ewm_h, outp_h,
                  sidx_all, didx_all, ew_all, rows0, rows1, acc,
                  semg0, semg1):
    cid = lax.axis_index("c")
    sid = lax.axis_index("s")
    z16 = _zero16()

    def zr(r, carry):
        for c in range(8):
            rows0[r, pl.ds(c * 16, 16)] = z16
        return carry

    lax.fori_loop(0, SW, zr, 0)
    for k in range(ROWS_PER_TILE // SW):
        pltpu.sync_copy(rows0,
                        acc.at[pl.ds(sid * ROWS_PER_TILE + k * SW, SW)])
    ebase = (cid * NT + sid) * ET
    pltpu.sync_copy(src_h.at[pl.ds(ebase, ET)], sidx_all)
    pltpu.sync_copy(dst_h.at[pl.ds(ebase, ET)], didx_all)
    pltpu.sync_copy(ewm_h.at[pl.ds(ebase, ET)], ew_all)
    plsc.subcore_barrier()

    def scale_scatter(w, buf):
        base = jnp.zeros((16,), jnp.int32) + w * SW

        def scale(e, c2):
            b = plsc.load_gather(ew_all, [base + e])
            for c in range(8):
                buf[e, pl.ds(c * 16, 16)] = buf[e, pl.ds(c * 16, 16)] * b
            return c2

        lax.fori_loop(0, SW, scale, 0)
        pltpu.sync_copy(buf, acc.at[didx_all.at[pl.ds(w * SW, SW)]],
                        add=True)

    def gat(w, buf, sem):
        pltpu.async_copy(hwp_h.at[sidx_all.at[pl.ds(w * SW, SW)]],
                         buf, sem)

    def gat_wait(w, buf, sem):
        pltpu.make_async_copy(hwp_h.at[sidx_all.at[pl.ds(w * SW, SW)]],
                              buf, sem).wait()

    gat(0, rows0, semg0)

    def pair(p, carry):
        w0 = 2 * p
        gat_wait(w0, rows0, semg0)
        gat(w0 + 1, rows1, semg1)
        scale_scatter(w0, rows0)
        gat(w0 + 2, rows0, semg0)
        gat_wait(w0 + 1, rows1, semg1)
        scale_scatter(w0 + 1, rows1)
        return carry

    lax.fori_loop(0, NW2 // 2 - 1, pair, 0)
    gat_wait(NW2 - 2, rows0, semg0)
    gat(NW2 - 1, rows1, semg1)
    scale_scatter(NW2 - 2, rows0)
    gat_wait(NW2 - 1, rows1, semg1)
    scale_scatter(NW2 - 1, rows1)
    plsc.subcore_barrier()
    for k in range(ROWS_PER_TILE // SW):
        r0 = sid * ROWS_PER_TILE + k * SW
        pltpu.sync_copy(acc.at[pl.ds(r0, SW)],
                        outp_h.at[cid, pl.ds(r0, SW)])


_sc_prop = functools.partial(
    pl.kernel,
    out_type=[jax.ShapeDtypeStruct((NC, NPAD, D), jnp.float32)],
    mesh=_mesh,
    compiler_params=pltpu.CompilerParams(needs_layout_passes=False),
    scratch_types=[
        pltpu.VMEM((ET,), jnp.int32),
        pltpu.VMEM((ET,), jnp.int32),
        pltpu.VMEM((ET,), jnp.float32),
        pltpu.VMEM((SW, D), jnp.float32),
        pltpu.VMEM((SW, D), jnp.float32),
        pltpu.VMEM_SHARED((NPAD, D), jnp.float32),
        pltpu.SemaphoreType.DMA,
        pltpu.SemaphoreType.DMA,
    ],
)(_sc_prop_body)


def _sc_score_body(hw1_h, src_h, dst_h, ewm_h, scp_h,
                   hw1_v, sidx_all, didx_all, ew_all, u_all, zb, acc, sem):
    cid = lax.axis_index("c")
    sid = lax.axis_index("s")
    z16 = _zero16()
    for i in range(40):
        zb[pl.ds(i * 16, 16)] = z16
    pltpu.sync_copy(zb, acc.at[pl.ds(sid * 640, 640)])
    pltpu.sync_copy(hw1_h, hw1_v)
    wbase = (cid * NT + sid) * NWIN
    pltpu.sync_copy(src_h.at[pl.ds(wbase, NWIN)], sidx_all)
    pltpu.sync_copy(dst_h.at[pl.ds(wbase, NWIN)], didx_all)
    pltpu.sync_copy(ewm_h.at[pl.ds(wbase, NWIN)], ew_all)
    plsc.subcore_barrier()

    def win(w, carry):
        for k in range(W // 16):
            sv = sidx_all[w, 0, pl.ds(k * 16, 16)]
            hv = plsc.load_gather(hw1_v, [sv])
            u_all[w, 0, pl.ds(k * 16, 16)] = \
                ew_all[w, 0, pl.ds(k * 16, 16)] * hv
        return carry

    lax.fori_loop(0, NWIN, win, 0)

    def sc(w, carry):
        pltpu.sync_copy(u_all.at[w, 0], acc.at[didx_all.at[w, 0]],
                        add=True)
        return carry

    lax.fori_loop(0, NWIN, sc, 0)
    plsc.subcore_barrier()
    pltpu.sync_copy(acc.at[pl.ds(sid * 640, 640)],
                    scp_h.at[cid, pl.ds(sid * 640, 640)])


_sc_score = functools.partial(
    pl.kernel,
    out_type=[jax.ShapeDtypeStruct((NC, NPAD), jnp.float32)],
    mesh=_mesh,
    compiler_params=pltpu.CompilerParams(needs_layout_passes=False),
    scratch_types=[
        pltpu.VMEM((NPAD,), jnp.float32),
        pltpu.VMEM((NWIN, 1, W), jnp.int32),
        pltpu.VMEM((NWIN, 1, W), jnp.int32),
        pltpu.VMEM((NWIN, 1, W), jnp.float32),
        pltpu.VMEM((NWIN, 1, W), jnp.float32),
        pltpu.VMEM((640,), jnp.float32),
        pltpu.VMEM_SHARED((NPAD,), jnp.float32),
        pltpu.SemaphoreType.DMA,
    ],
)(_sc_score_body)


def _tc_mm_scale_body(a_ref, w_ref, degp_ref, o_ref, dinv_ref):
    deg = degp_ref[0] + degp_ref[1] + 1.0
    dinv = jnp.where(deg > 0, lax.rsqrt(jnp.maximum(deg, 1e-12)), 0.0)
    dinv_ref[...] = dinv
    o_ref[...] = jnp.dot(a_ref[...], w_ref[...],
                         preferred_element_type=jnp.float32) * dinv


def _tc_mm_scale(h, w, degp2):
    return pl.pallas_call(
        _tc_mm_scale_body,
        grid=(NBLK,),
        in_specs=[
            pl.BlockSpec((BR, D), lambda i: (i, 0)),
            pl.BlockSpec((D, D), lambda i: (0, 0)),
            pl.BlockSpec((NC, BR, 1), lambda i: (0, i, 0)),
        ],
        out_specs=[
            pl.BlockSpec((BR, D), lambda i: (i, 0)),
            pl.BlockSpec((BR, 1), lambda i: (i, 0)),
        ],
        out_shape=[
            jax.ShapeDtypeStruct((NPAD, D), jnp.float32),
            jax.ShapeDtypeStruct((NPAD, 1), jnp.float32),
        ],
    )(h, w, degp2)


def _tc_post_body(outp_ref, hwp_ref, dinv_ref, b_ref, ws_ref,
                  h2_ref, hw1_ref):
    s = outp_ref[0] + outp_ref[1] + hwp_ref[...]
    h2 = jax.nn.relu(s * dinv_ref[...] + b_ref[...])
    h2_ref[...] = h2
    hw1_ref[...] = jnp.sum(h2 * ws_ref[...], axis=1,
                           keepdims=True) * dinv_ref[...]


def _tc_post(outp, hwp, dinv, b, ws_row):
    return pl.pallas_call(
        _tc_post_body,
        grid=(NBLK,),
        in_specs=[
            pl.BlockSpec((NC, BR, D), lambda i: (0, i, 0)),
            pl.BlockSpec((BR, D), lambda i: (i, 0)),
            pl.BlockSpec((BR, 1), lambda i: (i, 0)),
            pl.BlockSpec((1, D), lambda i: (0, 0)),
            pl.BlockSpec((1, D), lambda i: (0, 0)),
        ],
        out_specs=[
            pl.BlockSpec((BR, D), lambda i: (i, 0)),
            pl.BlockSpec((BR, 1), lambda i: (i, 0)),
        ],
        out_shape=[
            jax.ShapeDtypeStruct((NPAD, D), jnp.float32),
            jax.ShapeDtypeStruct((NPAD, 1), jnp.float32),
        ],
    )(outp, hwp, dinv, b, ws_row)


def _tc_spost_body(scp_ref, hw1_ref, dinv_ref, bs_ref, score_ref):
    pre = (scp_ref[0] + scp_ref[1] + hw1_ref[...]) * dinv_ref[...] \
        + bs_ref[...]
    score_ref[...] = jnp.tanh(pre)


def _tc_spost(scp2, hw1, dinv, bs):
    return pl.pallas_call(
        _tc_spost_body,
        grid=(NBLK,),
        in_specs=[
            pl.BlockSpec((NC, BR, 1), lambda i: (0, i, 0)),
            pl.BlockSpec((BR, 1), lambda i: (i, 0)),
            pl.BlockSpec((BR, 1), lambda i: (i, 0)),
            pl.BlockSpec((1, 1), lambda i: (0, 0)),
        ],
        out_specs=pl.BlockSpec((BR, 1), lambda i: (i, 0)),
        out_shape=jax.ShapeDtypeStruct((NPAD, 1), jnp.float32),
    )(scp2, hw1, dinv, bs)


def _tc_readout_body(h2_ref, score_ref, kf_ref, gid_ref,
                     hn_ref, mean_ref, mx_ref, cnt_ref):
    i = pl.program_id(0)
    hn = h2_ref[...] * score_ref[...] * kf_ref[...]
    hn_ref[...] = hn

    @pl.when(i == 0)
    def _():
        mean_ref[...] = jnp.zeros((G, D), jnp.float32)
        mx_ref[...] = jnp.full((G, D), -1e30, jnp.float32)
        cnt_ref[...] = jnp.zeros((G, D), jnp.float32)

    gid = gid_ref[...]
    kf = kf_ref[...]
    gmin = jnp.min(gid)
    gmax = jnp.max(gid)
    for g in range(G):
        @pl.when((gmin <= g) & (g <= gmax))
        def _():
            m = gid == g
            msum = jnp.sum(jnp.where(m, hn, 0.0), axis=0, keepdims=True)
            mean_ref[pl.ds(g, 1), :] += msum
            kept = m & (kf > 0)
            mmax = jnp.max(jnp.where(kept, hn, -1e30), axis=0, keepdims=True)
            mx_ref[pl.ds(g, 1), :] = jnp.maximum(mx_ref[pl.ds(g, 1), :], mmax)
            c = jnp.sum(jnp.where(m, kf, 0.0))
            cnt_ref[pl.ds(g, 1), :] += jnp.full((1, D), 1.0,
                                                jnp.float32) * c

    @pl.when(i == NBLK - 1)
    def _():
        cnt = cnt_ref[...]
        mean_ref[...] = mean_ref[...] / jnp.maximum(cnt, 1.0)
        mx_ref[...] = jnp.where(cnt > 0, mx_ref[...], 0.0)


def _tc_readout(h2, score, kf_col, gid_col):
    return pl.pallas_call(
        _tc_readout_body,
        grid=(NBLK,),
        in_specs=[
            pl.BlockSpec((BR, D), lambda i: (i, 0)),
            pl.BlockSpec((BR, 1), lambda i: (i, 0)),
            pl.BlockSpec((BR, 1), lambda i: (i, 0)),
            pl.BlockSpec((BR, 1), lambda i: (i, 0)),
        ],
        out_specs=[
            pl.BlockSpec((BR, D), lambda i: (i, 0)),
            pl.BlockSpec((G, D), lambda i: (0, 0)),
            pl.BlockSpec((G, D), lambda i: (0, 0)),
            pl.BlockSpec((G, D), lambda i: (0, 0)),
        ],
        out_shape=[
            jax.ShapeDtypeStruct((NPAD, D), jnp.float32),
            jax.ShapeDtypeStruct((G, D), jnp.float32),
            jax.ShapeDtypeStruct((G, D), jnp.float32),
            jax.ShapeDtypeStruct((G, D), jnp.float32),
        ],
    )(h2, score, kf_col, gid_col)


def _tc_mlp_body(ms_ref, xs_ref, w1a_ref, w1b_ref, b1_ref,
                 w2_ref, b2_ref, w3_ref, b3_ref, o_ref):
    z1 = jax.nn.relu(
        jnp.dot(ms_ref[...], w1a_ref[...], preferred_element_type=jnp.float32)
        + jnp.dot(xs_ref[...], w1b_ref[...],
                  preferred_element_type=jnp.float32)
        + b1_ref[...])
    z2 = jax.nn.relu(
        jnp.dot(z1, w2_ref[...], preferred_element_type=jnp.float32)
        + b2_ref[...])
    o_ref[...] = jnp.dot(z2, w3_ref[...],
                         preferred_element_type=jnp.float32) + b3_ref[...]


def _tc_mlp(ms, xs, w1a, w1b, b1, w2, b2, w3p, b3p):
    return pl.pallas_call(
        _tc_mlp_body,
        out_shape=jax.ShapeDtypeStruct((G, D), jnp.float32),
    )(ms, xs, w1a, w1b, b1, w2, b2, w3p, b3p)


CW = 128


def _tc_topk_body(sr_ref, kr_ref, gr_ref, sc_ref, kc_ref, gc_ref,
                  o_ref, rank_ref, cnt_ref):
    i = pl.program_id(0)
    sr = sr_ref[...]
    kr = kr_ref[...]
    gr = gr_ref[...]
    rmin = jnp.min(gr)
    rmax = jnp.max(gr)
    rank_ref[...] = jnp.zeros((BR, 1), jnp.float32)
    cnt_ref[...] = jnp.zeros((BR, 1), jnp.float32)
    ridx = i * BR + lax.broadcasted_iota(jnp.int32, (BR, 1), 0)
    for c in range(NPAD // CW):
        gc = gc_ref[:, pl.ds(c * CW, CW)]
        cmin = jnp.min(gc)
        cmax = jnp.max(gc)

        @pl.when((cmin <= rmax) & (cmax >= rmin))
        def _():
            scv = sc_ref[:, pl.ds(c * CW, CW)]
            kcv = kc_ref[:, pl.ds(c * CW, CW)]
            cidx = c * CW + lax.broadcasted_iota(jnp.int32, (1, CW), 1)
            mb = jnp.where((gc == gr) & (kcv > 0), 1.0, 0.0)
            hi = (scv > sr) | ((scv == sr) & (cidx < ridx))
            cnt_ref[...] += jnp.sum(mb, axis=1, keepdims=True)
            rank_ref[...] += jnp.sum(jnp.where(hi, mb, 0.0), axis=1,
                                     keepdims=True)

    kper = jnp.ceil(RATIO * cnt_ref[...])
    o_ref[...] = jnp.where((kr > 0) & (rank_ref[...] < kper), 1.0, 0.0)


def _tc_topk(score, kfc, gidc, score_t, kf_t, gid_t):
    return pl.pallas_call(
        _tc_topk_body,
        grid=(NBLK,),
        in_specs=[
            pl.BlockSpec((BR, 1), lambda i: (i, 0)),
            pl.BlockSpec((BR, 1), lambda i: (i, 0)),
            pl.BlockSpec((BR, 1), lambda i: (i, 0)),
            pl.BlockSpec((1, NPAD), lambda i: (0, 0)),
            pl.BlockSpec((1, NPAD), lambda i: (0, 0)),
            pl.BlockSpec((1, NPAD), lambda i: (0, 0)),
        ],
        out_specs=pl.BlockSpec((BR, 1), lambda i: (i, 0)),
        out_shape=jax.ShapeDtypeStruct((NPAD, 1), jnp.float32),
        scratch_shapes=[
            pltpu.VMEM((BR, 1), jnp.float32),
            pltpu.VMEM((BR, 1), jnp.float32),
        ],
    )(score, kfc, gidc, score_t, kf_t, gid_t)


def kernel(x, edge_index, edge_weight, node_graph_index,
           W_gcn0, b_gcn0, W_s0, b_s0,
           W_gcn1, b_gcn1, W_s1, b_s1,
           W_gcn2, b_gcn2, W_s2, b_s2,
           W_m1, b_m1, W_m2, b_m2, W_m3, b_m3):
    src = edge_index[0].reshape(E // W, 1, W)
    dst = edge_index[1].reshape(E // W, 1, W)
    gid = node_graph_index

    h = jnp.pad(x, ((0, NPAD - N), (0, 0)))
    gid_col = jnp.pad(gid, (0, NPAD - N), constant_values=G).reshape(NPAD, 1)
    kfc = jnp.pad(jnp.ones((N,), jnp.float32),
                  (0, NPAD - N)).reshape(NPAD, 1)
    ew = edge_weight.reshape(E // W, 1, W)

    means = []
    maxes = []
    for Wg, bg, Ws, bs in (
        (W_gcn0, b_gcn0, W_s0, b_s0),
        (W_gcn1, b_gcn1, W_s1, b_s1),
        (W_gcn2, b_gcn2, W_s2, b_s2),
    ):
        ewm, degp = _sc_mask_deg(src, dst, ew, kfc.reshape(NPAD))
        hwp, dinv = _tc_mm_scale(h, Wg, degp.reshape(NC, NPAD, 1))
        outp, = _sc_prop(hwp, edge_index[0], edge_index[1],
                         ewm.reshape(E))
        h2, hw1 = _tc_post(outp, hwp, dinv, bg.reshape(1, D),
                           Ws.reshape(1, D))
        scp, = _sc_score(hw1.reshape(NPAD), src, dst, ewm)
        score = _tc_spost(scp.reshape(NC, NPAD, 1), hw1, dinv,
                          bs.reshape(1, 1))

        kfc = _tc_topk(score, kfc, gid_col,
                       score.reshape(1, NPAD), kfc.reshape(1, NPAD),
                       gid_col.reshape(1, NPAD))
        hn, mean, mx, _cnt = _tc_readout(h2, score, kfc, gid_col)
        means.append(mean)
        maxes.append(mx)
        h = hn
        ew = ewm

    w3p = jnp.pad(W_m3, ((0, 0), (0, D - W_m3.shape[1])))
    b3p = jnp.pad(b_m3, (0, D - b_m3.shape[0])).reshape(1, D)
    out = _tc_mlp(means[0] + means[1] + means[2],
                  maxes[0] + maxes[1] + maxes[2],
                  W_m1[:D], W_m1[D:], b_m1.reshape(1, D),
                  W_m2, b_m2.reshape(1, W_m2.shape[1]),
                  w3p, b3p)
    return out[:, :W_m3.shape[1]]


# speedup vs baseline: 1.5260x; 1.5260x over previous
import functools

import jax
import jax.numpy as jnp
from jax import lax
from jax.experimental import pallas as pl
from jax.experimental.pallas import tpu as pltpu
from jax.experimental.pallas import tpu_sc as plsc

N = 10000
E = 320000
D = 128
G = 64
RATIO = 0.5

NPAD = 10240
NC = 2
NT = 16
ET = E // (NC * NT)
W = 80
NWIN = ET // W
ROWS_PER_TILE = NPAD // NT

BR = 1280
NBLK = NPAD // BR

_mesh = plsc.VectorSubcoreMesh(core_axis_name="c", subcore_axis_name="s",
                               num_cores=NC)


def _zero16():
    return jnp.zeros((16,), jnp.float32)


def _sc_mask_deg_body(src_h, dst_h, ew_h, kf_h, ewm_h, degp_h,
                      kf_v, sidx_all, didx_all, ew_all, ewm_all, zb, acc, sem):
    cid = lax.axis_index("c")
    sid = lax.axis_index("s")
    z16 = _zero16()
    for i in range(40):
        zb[pl.ds(i * 16, 16)] = z16
    pltpu.sync_copy(zb, acc.at[pl.ds(sid * 640, 640)])
    pltpu.sync_copy(kf_h, kf_v)
    wbase = (cid * NT + sid) * NWIN
    pltpu.sync_copy(src_h.at[pl.ds(wbase, NWIN)], sidx_all)
    pltpu.sync_copy(dst_h.at[pl.ds(wbase, NWIN)], didx_all)
    pltpu.sync_copy(ew_h.at[pl.ds(wbase, NWIN)], ew_all)
    plsc.subcore_barrier()

    def win(w, carry):
        for k in range(W // 16):
            sv = sidx_all[w, 0, pl.ds(k * 16, 16)]
            dv = didx_all[w, 0, pl.ds(k * 16, 16)]
            ks = plsc.load_gather(kf_v, [sv])
            kd = plsc.load_gather(kf_v, [dv])
            ewm_all[w, 0, pl.ds(k * 16, 16)] = \
                ew_all[w, 0, pl.ds(k * 16, 16)] * ks * kd
        return carry

    lax.fori_loop(0, NWIN, win, 0)
    pltpu.sync_copy(ewm_all, ewm_h.at[pl.ds(wbase, NWIN)])

    def sc(w, carry):
        pltpu.sync_copy(ewm_all.at[w, 0], acc.at[didx_all.at[w, 0]],
                        add=True)
        return carry

    lax.fori_loop(0, NWIN, sc, 0)
    plsc.subcore_barrier()
    pltpu.sync_copy(acc.at[pl.ds(sid * 640, 640)],
                    degp_h.at[cid, pl.ds(sid * 640, 640)])


_sc_mask_deg = functools.partial(
    pl.kernel,
    out_type=[jax.ShapeDtypeStruct((E // W, 1, W), jnp.float32),
              jax.ShapeDtypeStruct((NC, NPAD), jnp.float32)],
    mesh=_mesh,
    compiler_params=pltpu.CompilerParams(needs_layout_passes=False),
    scratch_types=[
        pltpu.VMEM((NPAD,), jnp.float32),
        pltpu.VMEM((NWIN, 1, W), jnp.int32),
        pltpu.VMEM((NWIN, 1, W), jnp.int32),
        pltpu.VMEM((NWIN, 1, W), jnp.float32),
        pltpu.VMEM((NWIN, 1, W), jnp.float32),
        pltpu.VMEM((640,), jnp.float32),
        pltpu.VMEM_SHARED((NPAD,), jnp.float32),
        pltpu.SemaphoreType.DMA,
    ],
)(_sc_mask_deg_body)


SW = 40
NW2 = ET // SW


def _sc_prop_body(hwp_h, src_h, dst_h, ewm_h, outp_h,
                  sidx_all, didx_all, ew_all, rows0, rows1, acc,
                  semg0, semg1):
    cid = lax.axis_index("c")
    sid = lax.axis_index("s")
    z16 = _zero16()

    def zr(r, carry):
        for c in range(8):
            rows0[r, pl.ds(c * 16, 16)] = z16
        return carry

    lax.fori_loop(0, SW, zr, 0)
    for k in range(ROWS_PER_TILE // SW):
        pltpu.sync_copy(rows0,
                        acc.at[pl.ds(sid * ROWS_PER_TILE + k * SW, SW)])
    ebase = (cid * NT + sid) * ET
    pltpu.sync_copy(src_h.at[pl.ds(ebase, ET)], sidx_all)
    pltpu.sync_copy(dst_h.at[pl.ds(ebase, ET)], didx_all)
    pltpu.sync_copy(ewm_h.at[pl.ds(ebase, ET)], ew_all)
    plsc.subcore_barrier()

    def scale_scatter(w, buf):
        base = jnp.zeros((16,), jnp.int32) + w * SW

        def scale(e, c2):
            b = plsc.load_gather(ew_all, [base + e])
            for c in range(8):
                buf[e, pl.ds(c * 16, 16)] = buf[e, pl.ds(c * 16, 16)] * b
            return c2

        lax.fori_loop(0, SW, scale, 0)
        pltpu.sync_copy(buf, acc.at[didx_all.at[pl.ds(w * SW, SW)]],
                        add=True)

    def gat(w, buf, sem):
        pltpu.async_copy(hwp_h.at[sidx_all.at[pl.ds(w * SW, SW)]],
                         buf, sem)

    def gat_wait(w, buf, sem):
        pltpu.make_async_copy(hwp_h.at[sidx_all.at[pl.ds(w * SW, SW)]],
                              buf, sem).wait()

    gat(0, rows0, semg0)

    def pair(p, carry):
        w0 = 2 * p
        gat_wait(w0, rows0, semg0)
        gat(w0 + 1, rows1, semg1)
        scale_scatter(w0, rows0)
        gat(w0 + 2, rows0, semg0)
        gat_wait(w0 + 1, rows1, semg1)
        scale_scatter(w0 + 1, rows1)
        return carry

    lax.fori_loop(0, NW2 // 2 - 1, pair, 0)
    gat_wait(NW2 - 2, rows0, semg0)
    gat(NW2 - 1, rows1, semg1)
    scale_scatter(NW2 - 2, rows0)
    gat_wait(NW2 - 1, rows1, semg1)
    scale_scatter(NW2 - 1, rows1)
    plsc.subcore_barrier()
    for k in range(ROWS_PER_TILE // SW):
        r0 = sid * ROWS_PER_TILE + k * SW
        pltpu.sync_copy(acc.at[pl.ds(r0, SW)],
                        outp_h.at[cid, pl.ds(r0, SW)])


_sc_prop = functools.partial(
    pl.kernel,
    out_type=[jax.ShapeDtypeStruct((NC, NPAD, D), jnp.float32)],
    mesh=_mesh,
    compiler_params=pltpu.CompilerParams(needs_layout_passes=False),
    scratch_types=[
        pltpu.VMEM((ET,), jnp.int32),
        pltpu.VMEM((ET,), jnp.int32),
        pltpu.VMEM((ET,), jnp.float32),
        pltpu.VMEM((SW, D), jnp.float32),
        pltpu.VMEM((SW, D), jnp.float32),
        pltpu.VMEM_SHARED((NPAD, D), jnp.float32),
        pltpu.SemaphoreType.DMA,
        pltpu.SemaphoreType.DMA,
    ],
)(_sc_prop_body)


def _sc_score_body(hw1_h, src_h, dst_h, ewm_h, scp_h,
                   hw1_v, sidx_all, didx_all, ew_all, u_all, zb, acc, sem):
    cid = lax.axis_index("c")
    sid = lax.axis_index("s")
    z16 = _zero16()
    for i in range(40):
        zb[pl.ds(i * 16, 16)] = z16
    pltpu.sync_copy(zb, acc.at[pl.ds(sid * 640, 640)])
    pltpu.sync_copy(hw1_h, hw1_v)
    wbase = (cid * NT + sid) * NWIN
    pltpu.sync_copy(src_h.at[pl.ds(wbase, NWIN)], sidx_all)
    pltpu.sync_copy(dst_h.at[pl.ds(wbase, NWIN)], didx_all)
    pltpu.sync_copy(ewm_h.at[pl.ds(wbase, NWIN)], ew_all)
    plsc.subcore_barrier()

    def win(w, carry):
        for k in range(W // 16):
            sv = sidx_all[w, 0, pl.ds(k * 16, 16)]
            hv = plsc.load_gather(hw1_v, [sv])
            u_all[w, 0, pl.ds(k * 16, 16)] = \
                ew_all[w, 0, pl.ds(k * 16, 16)] * hv
        return carry

    lax.fori_loop(0, NWIN, win, 0)

    def sc(w, carry):
        pltpu.sync_copy(u_all.at[w, 0], acc.at[didx_all.at[w, 0]],
                        add=True)
        return carry

    lax.fori_loop(0, NWIN, sc, 0)
    plsc.subcore_barrier()
    pltpu.sync_copy(acc.at[pl.ds(sid * 640, 640)],
                    scp_h.at[cid, pl.ds(sid * 640, 640)])


_sc_score = functools.partial(
    pl.kernel,
    out_type=[jax.ShapeDtypeStruct((NC, NPAD), jnp.float32)],
    mesh=_mesh,
    compiler_params=pltpu.CompilerParams(needs_layout_passes=False),
    scratch_types=[
        pltpu.VMEM((NPAD,), jnp.float32),
        pltpu.VMEM((NWIN, 1, W), jnp.int32),
        pltpu.VMEM((NWIN, 1, W), jnp.int32),
        pltpu.VMEM((NWIN, 1, W), jnp.float32),
        pltpu.VMEM((NWIN, 1, W), jnp.float32),
        pltpu.VMEM((640,), jnp.float32),
        pltpu.VMEM_SHARED((NPAD,), jnp.float32),
        pltpu.SemaphoreType.DMA,
    ],
)(_sc_score_body)


def _tc_mm_scale_body(a_ref, w_ref, degp_ref, o_ref, dinv_ref):
    deg = degp_ref[0] + degp_ref[1] + 1.0
    dinv = jnp.where(deg > 0, lax.rsqrt(jnp.maximum(deg, 1e-12)), 0.0)
    dinv_ref[...] = dinv
    o_ref[...] = jnp.dot(a_ref[...], w_ref[...],
                         preferred_element_type=jnp.float32) * dinv


def _tc_mm_scale(h, w, degp2):
    return pl.pallas_call(
        _tc_mm_scale_body,
        grid=(NBLK,),
        in_specs=[
            pl.BlockSpec((BR, D), lambda i: (i, 0)),
            pl.BlockSpec((D, D), lambda i: (0, 0)),
            pl.BlockSpec((NC, BR, 1), lambda i: (0, i, 0)),
        ],
        out_specs=[
            pl.BlockSpec((BR, D), lambda i: (i, 0)),
            pl.BlockSpec((BR, 1), lambda i: (i, 0)),
        ],
        out_shape=[
            jax.ShapeDtypeStruct((NPAD, D), jnp.float32),
            jax.ShapeDtypeStruct((NPAD, 1), jnp.float32),
        ],
    )(h, w, degp2)


def _tc_post_body(outp_ref, hwp_ref, dinv_ref, b_ref, ws_ref,
                  h2_ref, hw1_ref):
    s = outp_ref[0] + outp_ref[1] + hwp_ref[...]
    h2 = jax.nn.relu(s * dinv_ref[...] + b_ref[...])
    h2_ref[...] = h2
    hw1_ref[...] = jnp.sum(h2 * ws_ref[...], axis=1,
                           keepdims=True) * dinv_ref[...]


def _tc_post(outp, hwp, dinv, b, ws_row):
    return pl.pallas_call(
        _tc_post_body,
        grid=(NBLK,),
        in_specs=[
            pl.BlockSpec((NC, BR, D), lambda i: (0, i, 0)),
            pl.BlockSpec((BR, D), lambda i: (i, 0)),
            pl.BlockSpec((BR, 1), lambda i: (i, 0)),
            pl.BlockSpec((1, D), lambda i: (0, 0)),
            pl.BlockSpec((1, D), lambda i: (0, 0)),
        ],
        out_specs=[
            pl.BlockSpec((BR, D), lambda i: (i, 0)),
            pl.BlockSpec((BR, 1), lambda i: (i, 0)),
        ],
        out_shape=[
            jax.ShapeDtypeStruct((NPAD, D), jnp.float32),
            jax.ShapeDtypeStruct((NPAD, 1), jnp.float32),
        ],
    )(outp, hwp, dinv, b, ws_row)


def _tc_spost_body(scp_ref, hw1_ref, dinv_ref, bs_ref, score_ref):
    pre = (scp_ref[0] + scp_ref[1] + hw1_ref[...]) * dinv_ref[...] \
        + bs_ref[...]
    score_ref[...] = jnp.tanh(pre)


def _tc_spost(scp2, hw1, dinv, bs):
    return pl.pallas_call(
        _tc_spost_body,
        grid=(NBLK,),
        in_specs=[
            pl.BlockSpec((NC, BR, 1), lambda i: (0, i, 0)),
            pl.BlockSpec((BR, 1), lambda i: (i, 0)),
            pl.BlockSpec((BR, 1), lambda i: (i, 0)),
            pl.BlockSpec((1, 1), lambda i: (0, 0)),
        ],
        out_specs=pl.BlockSpec((BR, 1), lambda i: (i, 0)),
        out_shape=jax.ShapeDtypeStruct((NPAD, 1), jnp.float32),
    )(scp2, hw1, dinv, bs)


def _tc_readout_body(h2_ref, score_ref, kf_ref, gid_ref,
                     hn_ref, mean_ref, mx_ref, cnt_ref):
    i = pl.program_id(0)
    hn = h2_ref[...] * score_ref[...] * kf_ref[...]
    hn_ref[...] = hn

    @pl.when(i == 0)
    def _():
        mean_ref[...] = jnp.zeros((G, D), jnp.float32)
        mx_ref[...] = jnp.full((G, D), -1e30, jnp.float32)
        cnt_ref[...] = jnp.zeros((G, D), jnp.float32)

    gid = gid_ref[...]
    kf = kf_ref[...]
    gmin = jnp.min(gid)
    gmax = jnp.max(gid)
    for g in range(G):
        @pl.when((gmin <= g) & (g <= gmax))
        def _():
            m = gid == g
            msum = jnp.sum(jnp.where(m, hn, 0.0), axis=0, keepdims=True)
            mean_ref[pl.ds(g, 1), :] += msum
            kept = m & (kf > 0)
            mmax = jnp.max(jnp.where(kept, hn, -1e30), axis=0, keepdims=True)
            mx_ref[pl.ds(g, 1), :] = jnp.maximum(mx_ref[pl.ds(g, 1), :], mmax)
            c = jnp.sum(jnp.where(m, kf, 0.0))
            cnt_ref[pl.ds(g, 1), :] += jnp.full((1, D), 1.0,
                                                jnp.float32) * c

    @pl.when(i == NBLK - 1)
    def _():
        cnt = cnt_ref[...]
        mean_ref[...] = mean_ref[...] / jnp.maximum(cnt, 1.0)
        mx_ref[...] = jnp.where(cnt > 0, mx_ref[...], 0.0)


def _tc_readout(h2, score, kf_col, gid_col):
    return pl.pallas_call(
        _tc_readout_body,
        grid=(NBLK,),
        in_specs=[
            pl.BlockSpec((BR, D), lambda i: (i, 0)),
            pl.BlockSpec((BR, 1), lambda i: (i, 0)),
            pl.BlockSpec((BR, 1), lambda i: (i, 0)),
            pl.BlockSpec((BR, 1), lambda i: (i, 0)),
        ],
        out_specs=[
            pl.BlockSpec((BR, D), lambda i: (i, 0)),
            pl.BlockSpec((G, D), lambda i: (0, 0)),
            pl.BlockSpec((G, D), lambda i: (0, 0)),
            pl.BlockSpec((G, D), lambda i: (0, 0)),
        ],
        out_shape=[
            jax.ShapeDtypeStruct((NPAD, D), jnp.float32),
            jax.ShapeDtypeStruct((G, D), jnp.float32),
            jax.ShapeDtypeStruct((G, D), jnp.float32),
            jax.ShapeDtypeStruct((G, D), jnp.float32),
        ],
    )(h2, score, kf_col, gid_col)


def _tc_mlp_body(ms_ref, xs_ref, w1a_ref, w1b_ref, b1_ref,
                 w2_ref, b2_ref, w3_ref, b3_ref, o_ref):
    z1 = jax.nn.relu(
        jnp.dot(ms_ref[...], w1a_ref[...], preferred_element_type=jnp.float32)
        + jnp.dot(xs_ref[...], w1b_ref[...],
                  preferred_element_type=jnp.float32)
        + b1_ref[...])
    z2 = jax.nn.relu(
        jnp.dot(z1, w2_ref[...], preferred_element_type=jnp.float32)
        + b2_ref[...])
    o_ref[...] = jnp.dot(z2, w3_ref[...],
                         preferred_element_type=jnp.float32) + b3_ref[...]


def _tc_mlp(ms, xs, w1a, w1b, b1, w2, b2, w3p, b3p):
    return pl.pallas_call(
        _tc_mlp_body,
        out_shape=jax.ShapeDtypeStruct((G, D), jnp.float32),
    )(ms, xs, w1a, w1b, b1, w2, b2, w3p, b3p)


CW = 512


def _tc_topk_body(sr_ref, kr_ref, gr_ref, sc_ref, kc_ref, gc_ref,
                  o_ref, rank_ref, cnt_ref):
    i = pl.program_id(0)
    sr = sr_ref[...]
    kr = kr_ref[...]
    gr = gr_ref[...]
    rmin = jnp.min(gr)
    rmax = jnp.max(gr)
    rank_ref[...] = jnp.zeros((BR, 1), jnp.float32)
    cnt_ref[...] = jnp.zeros((BR, 1), jnp.float32)
    ridx = i * BR + lax.broadcasted_iota(jnp.int32, (BR, 1), 0)
    for c in range(NPAD // CW):
        gc = gc_ref[:, pl.ds(c * CW, CW)]
        cmin = jnp.min(gc)
        cmax = jnp.max(gc)

        @pl.when((cmin <= rmax) & (cmax >= rmin))
        def _():
            scv = sc_ref[:, pl.ds(c * CW, CW)]
            kcv = kc_ref[:, pl.ds(c * CW, CW)]
            cidx = c * CW + lax.broadcasted_iota(jnp.int32, (1, CW), 1)
            mb = jnp.where((gc == gr) & (kcv > 0), 1.0, 0.0)
            hi = (scv > sr) | ((scv == sr) & (cidx < ridx))
            cnt_ref[...] += jnp.sum(mb, axis=1, keepdims=True)
            rank_ref[...] += jnp.sum(jnp.where(hi, mb, 0.0), axis=1,
                                     keepdims=True)

    kper = jnp.ceil(RATIO * cnt_ref[...])
    o_ref[...] = jnp.where((kr > 0) & (rank_ref[...] < kper), 1.0, 0.0)


def _tc_topk(score, kfc, gidc, score_t, kf_t, gid_t):
    return pl.pallas_call(
        _tc_topk_body,
        grid=(NBLK,),
        in_specs=[
            pl.BlockSpec((BR, 1), lambda i: (i, 0)),
            pl.BlockSpec((BR, 1), lambda i: (i, 0)),
            pl.BlockSpec((BR, 1), lambda i: (i, 0)),
            pl.BlockSpec((1, NPAD), lambda i: (0, 0)),
            pl.BlockSpec((1, NPAD), lambda i: (0, 0)),
            pl.BlockSpec((1, NPAD), lambda i: (0, 0)),
        ],
        out_specs=pl.BlockSpec((BR, 1), lambda i: (i, 0)),
        out_shape=jax.ShapeDtypeStruct((NPAD, 1), jnp.float32),
        scratch_shapes=[
            pltpu.VMEM((BR, 1), jnp.float32),
            pltpu.VMEM((BR, 1), jnp.float32),
        ],
    )(score, kfc, gidc, score_t, kf_t, gid_t)


def kernel(x, edge_index, edge_weight, node_graph_index,
           W_gcn0, b_gcn0, W_s0, b_s0,
           W_gcn1, b_gcn1, W_s1, b_s1,
           W_gcn2, b_gcn2, W_s2, b_s2,
           W_m1, b_m1, W_m2, b_m2, W_m3, b_m3):
    src = edge_index[0].reshape(E // W, 1, W)
    dst = edge_index[1].reshape(E // W, 1, W)
    gid = node_graph_index

    h = jnp.pad(x, ((0, NPAD - N), (0, 0)))
    gid_col = jnp.pad(gid, (0, NPAD - N), constant_values=G).reshape(NPAD, 1)
    kfc = jnp.pad(jnp.ones((N,), jnp.float32),
                  (0, NPAD - N)).reshape(NPAD, 1)
    ew = edge_weight.reshape(E // W, 1, W)

    means = []
    maxes = []
    for Wg, bg, Ws, bs in (
        (W_gcn0, b_gcn0, W_s0, b_s0),
        (W_gcn1, b_gcn1, W_s1, b_s1),
        (W_gcn2, b_gcn2, W_s2, b_s2),
    ):
        ewm, degp = _sc_mask_deg(src, dst, ew, kfc.reshape(NPAD))
        hwp, dinv = _tc_mm_scale(h, Wg, degp.reshape(NC, NPAD, 1))
        outp, = _sc_prop(hwp, edge_index[0], edge_index[1],
                         ewm.reshape(E))
        h2, hw1 = _tc_post(outp, hwp, dinv, bg.reshape(1, D),
                           Ws.reshape(1, D))
        scp, = _sc_score(hw1.reshape(NPAD), src, dst, ewm)
        score = _tc_spost(scp.reshape(NC, NPAD, 1), hw1, dinv,
                          bs.reshape(1, 1))

        kfc = _tc_topk(score, kfc, gid_col,
                       score.reshape(1, NPAD), kfc.reshape(1, NPAD),
                       gid_col.reshape(1, NPAD))
        hn, mean, mx, _cnt = _tc_readout(h2, score, kfc, gid_col)
        means.append(mean)
        maxes.append(mx)
        h = hn
        ew = ewm

    w3p = jnp.pad(W_m3, ((0, 0), (0, D - W_m3.shape[1])))
    b3p = jnp.pad(b_m3, (0, D - b_m3.shape[0])).reshape(1, D)
    out = _tc_mlp(means[0] + means[1] + means[2],
                  maxes[0] + maxes[1] + maxes[2],
                  W_m1[:D], W_m1[D:], b_m1.reshape(1, D),
                  W_m2, b_m2.reshape(1, W_m2.shape[1]),
                  w3p, b3p)
    return out[:, :W_m3.shape[1]]
